# Initial kernel scaffold; baseline (speedup 1.0000x reference)
import jax
import jax.numpy as jnp
from jax.experimental import pallas as pl


def kernel(input_ids, counts, values, io_flags, positions, gas_fee, token_table, count_table, value_table, position_table, io_table, gas_table):
    raise NotImplementedError("write your pallas kernel here")


# speedup vs baseline: 1.4580x; 1.4580x over previous
import functools

import jax
import jax.numpy as jnp
from jax import lax
from jax.experimental import pallas as pl
from jax.experimental.pallas import tpu as pltpu
from jax.experimental.pallas import tpu_sc as plsc

B, L, D = 1024, 200, 128
N = B * L
NC, NS = 2, 16
NW = NC * NS
RPW = N // NW
C = 128
NCHUNK = RPW // C
SEG = D // 16

_mesh = plsc.VectorSubcoreMesh(core_axis_name="c", subcore_axis_name="s")


@functools.partial(
    pl.kernel,
    mesh=_mesh,
    out_type=jax.ShapeDtypeStruct((N, D), jnp.float32),
    scratch_types=[
        pltpu.VMEM((C,), jnp.int32),
        pltpu.VMEM((C,), jnp.int32),
        pltpu.VMEM((C,), jnp.int32),
        pltpu.VMEM((C,), jnp.int32),
        pltpu.VMEM((C,), jnp.int32),
        pltpu.VMEM((C,), jnp.int32),
        pltpu.VMEM((C, D), jnp.float32),
        pltpu.VMEM((C, D), jnp.float32),
        pltpu.VMEM((C, D), jnp.float32),
        pltpu.VMEM((C, D), jnp.float32),
        pltpu.VMEM((C, D), jnp.float32),
        pltpu.VMEM((C, D), jnp.float32),
        pltpu.SemaphoreType.DMA,
    ],
)
def _embed_sum(tok_t, cnt_t, val_t, io_t, pos_t, gas_t,
               itok, icnt, ival, iio, ipos, igas,
               out_hbm,
               vtok, vcnt, vval, vio, vpos, vgas,
               acc, tcnt, tval, tio, tpos, tgas, sem):
    wid = lax.axis_index("s") * NC + lax.axis_index("c")

    def chunk(g, carry):
        base = wid * RPW + g * C
        sl = pl.ds(base, C)
        pltpu.sync_copy(itok.at[sl], vtok)
        pltpu.sync_copy(icnt.at[sl], vcnt)
        pltpu.sync_copy(ival.at[sl], vval)
        pltpu.sync_copy(iio.at[sl], vio)
        pltpu.sync_copy(ipos.at[sl], vpos)
        pltpu.sync_copy(igas.at[sl], vgas)
        cps = [
            pltpu.async_copy(tok_t.at[vtok], acc, sem),
            pltpu.async_copy(cnt_t.at[vcnt], tcnt, sem),
            pltpu.async_copy(val_t.at[vval], tval, sem),
            pltpu.async_copy(io_t.at[vio], tio, sem),
            pltpu.async_copy(pos_t.at[vpos], tpos, sem),
            pltpu.async_copy(gas_t.at[vgas], tgas, sem),
        ]
        for cp in cps:
            cp.wait()

        def row(r, rcarry):
            for s in range(SEG):
                cs = pl.ds(s * 16, 16)
                acc[r, cs] = (acc[r, cs] + tcnt[r, cs] + tval[r, cs]
                              + tio[r, cs] + tpos[r, cs] + tgas[r, cs])
            return rcarry

        lax.fori_loop(0, C, row, 0)
        pltpu.sync_copy(acc, out_hbm.at[sl])
        return carry

    lax.fori_loop(0, NCHUNK, chunk, 0)


def kernel(input_ids, counts, values, io_flags, positions, gas_fee,
           token_table, count_table, value_table, position_table,
           io_table, gas_table):
    flat = lambda a: a.reshape(N).astype(jnp.int32)
    out = _embed_sum(token_table, count_table, value_table, io_table,
                     position_table, gas_table,
                     flat(input_ids), flat(counts), flat(values),
                     flat(io_flags), flat(positions), flat(gas_fee))
    return out.reshape(B, L, D)
